# baseline (speedup 1.0000x reference)
import functools

import jax
import jax.numpy as jnp
from jax import lax
from jax.experimental import pallas as pl
from jax.experimental.pallas import tpu as pltpu
from jax.experimental.pallas import tpu_sc as plsc

_CHUNK = 32


def kernel(x, weight):
    batch, seq = x.shape
    dim = weight.shape[1]
    info = plsc.get_sparse_core_info()
    nw = info.num_cores * info.num_subcores
    rows_per_w = seq // nw
    nchunk = rows_per_w // _CHUNK

    mesh = plsc.VectorSubcoreMesh(core_axis_name="c", subcore_axis_name="s")

    @functools.partial(
        pl.kernel,
        mesh=mesh,
        out_type=jax.ShapeDtypeStruct((batch, seq, dim), weight.dtype),
        scratch_types=[
            pltpu.VMEM_SHARED((info.num_subcores, 2, _CHUNK, dim), weight.dtype),
            pltpu.SemaphoreType.DMA((2,)),
            pltpu.SemaphoreType.DMA((2,)),
        ],
    )
    def _sc_bcast(w_hbm, o_hbm, shared, sem_r, sem_w):
        sid = lax.axis_index("s")
        wid = sid * info.num_cores + lax.axis_index("c")
        base = wid * rows_per_w

        def read(i, slot):
            return pltpu.make_async_copy(
                w_hbm.at[pl.ds(base + i * _CHUNK, _CHUNK), :],
                shared.at[sid, slot],
                sem_r.at[slot],
            )

        def write(i, slot, b):
            return pltpu.make_async_copy(
                shared.at[sid, slot],
                o_hbm.at[b, pl.ds(base + i * _CHUNK, _CHUNK), :],
                sem_w.at[slot],
            )

        read(0, 0).start()
        for i in range(nchunk):
            slot = i % 2
            read(i, slot).wait()
            for b in range(batch):
                write(i, slot, b).start()
            if i + 1 < nchunk:
                if i >= 1:
                    for b in range(batch):
                        write(i - 1, 1 - slot, b).wait()
                read(i + 1, 1 - slot).start()
        for i in (nchunk - 2, nchunk - 1):
            for b in range(batch):
                write(i, i % 2, b).wait()

    return _sc_bcast(weight)
